# baseline (speedup 1.0000x reference)
import functools

import jax
import jax.numpy as jnp
from jax import lax
from jax.experimental import pallas as pl
from jax.experimental.pallas import tpu as pltpu
from jax.experimental.pallas import tpu_sc as plsc

N_NODES = 10000
HALF = N_NODES // 2
D = 128
N_EDGES = 320000
NC = 2
NS = 16
NW = NC * NS
K = 128
EPW = 10240
NCHUNK = EPW // K
EPAD = NW * EPW
RPW = 632
RPW_LAST = N_NODES - 15 * RPW
ARW = 312
ARW_LAST = HALF - 15 * ARW
CAP = 3072
CAPC = CAP // K
WCH = 8
NWIN = CAPC // WCH
ROWBLK = 1000
GRID = N_NODES // ROWBLK

_mesh = plsc.VectorSubcoreMesh(core_axis_name="c", subcore_axis_name="s")
_sc_params = pltpu.CompilerParams(needs_layout_passes=False)


@functools.partial(
    pl.kernel,
    out_type=jax.ShapeDtypeStruct((NW, N_NODES), jnp.float32),
    mesh=_mesh,
    scratch_types=[
        pltpu.VMEM((NCHUNK, K), jnp.int32),
        pltpu.VMEM((NCHUNK, K), jnp.float32),
        pltpu.VMEM((N_NODES,), jnp.float32),
    ],
    compiler_params=_sc_params,
)
def _deg_kernel(dst_hbm, ew_hbm, out_hbm, dst_v, ew_v, deg_v):
    c = lax.axis_index("c")
    s = lax.axis_index("s")
    wid = s * NC + c
    pltpu.sync_copy(dst_hbm.at[wid], dst_v)
    pltpu.sync_copy(ew_hbm.at[wid], ew_v)

    def zero_body(i, _):
        deg_v[pl.ds(i * 16, 16)] = jnp.zeros((16,), jnp.float32)
        return 0

    lax.fori_loop(0, N_NODES // 16, zero_body, 0, unroll=8)

    def acc_body(i, _):
        j = i // (K // 16)
        t = (i % (K // 16)) * 16
        d16 = dst_v[j, pl.ds(t, 16)]
        w16 = ew_v[j, pl.ds(t, 16)]
        plsc.addupdate_scatter(deg_v, [d16], w16)
        return 0

    lax.fori_loop(0, EPW // 16, acc_body, 0, unroll=4)

    pltpu.sync_copy(deg_v, out_hbm.at[wid])


def _part_body(src_hbm, dst_hbm, ew_hbm, srcl_hbm, dstl_hbm, ewl_hbm,
               src_v, dst_v, ew_v, srcb, dstb, ewb):
    c = lax.axis_index("c")
    s = lax.axis_index("s")
    wid = s * NC + c
    pltpu.sync_copy(src_hbm.at[wid], src_v)
    pltpu.sync_copy(dst_hbm.at[wid], dst_v)
    pltpu.sync_copy(ew_hbm.at[wid], ew_v)

    zi = jnp.zeros((16,), jnp.int32)
    zf = jnp.zeros((16,), jnp.float32)

    def zero_body(i, _):
        sl = pl.ds(i * 16, 16)
        srcb[sl] = zi
        dstb[sl] = zi
        ewb[sl] = zf
        return 0

    lax.fori_loop(0, 4 * CAP // 16, zero_body, 0, unroll=8)

    def scan_body(i, cur):
        j = i // (K // 16)
        t = (i % (K // 16)) * 16
        s16 = src_v[j, pl.ds(t, 16)]
        d16 = dst_v[j, pl.ds(t, 16)]
        w16 = ew_v[j, pl.ds(t, 16)]
        sh = s16 >= HALF
        dh = d16 >= HALF
        new_cur = []
        for b in range(4):
            bs, bd = b // 2, b % 2
            m = ((sh == (bs == 1)) & (dh == (bd == 1)))
            cb = cur[b]
            mi = m.astype(jnp.int32)
            incl = plsc.cumsum(mi)
            pos = cb + incl - mi
            fpos = pos + b * CAP
            plsc.store_scatter(srcb, [fpos], s16 - bs * HALF, mask=m)
            plsc.store_scatter(dstb, [fpos], d16 - bd * HALF, mask=m)
            plsc.store_scatter(ewb, [fpos], w16, mask=m)
            cnt = jnp.sum(mi)
            new_cur.append(jnp.minimum(cb + cnt, CAP - 16))
        return tuple(new_cur)

    final_cur = lax.fori_loop(0, EPW // 16, scan_body, (0, 0, 0, 0))

    del final_cur
    for b in range(4):
        sl = pl.ds(b * CAP, CAP)
        pltpu.sync_copy(srcb.at[sl], srcl_hbm.at[wid, b])
        pltpu.sync_copy(dstb.at[sl], dstl_hbm.at[wid, b])
        pltpu.sync_copy(ewb.at[sl], ewl_hbm.at[wid, b])


def _make_part_kernel(interpret=False):
    return functools.partial(
        pl.kernel,
        out_type=(
            jax.ShapeDtypeStruct((NW, 4, CAP), jnp.int32),
            jax.ShapeDtypeStruct((NW, 4, CAP), jnp.int32),
            jax.ShapeDtypeStruct((NW, 4, CAP), jnp.float32),
        ),
        mesh=_mesh,
        scratch_types=[
            pltpu.VMEM((NCHUNK, K), jnp.int32),
            pltpu.VMEM((NCHUNK, K), jnp.int32),
            pltpu.VMEM((NCHUNK, K), jnp.float32),
            pltpu.VMEM((4 * CAP,), jnp.int32),
            pltpu.VMEM((4 * CAP,), jnp.int32),
            pltpu.VMEM((4 * CAP,), jnp.float32),
        ],
        compiler_params=_sc_params,
        interpret=interpret,
    )(_part_body)


_part_kernel = _make_part_kernel()


@functools.partial(
    pl.kernel,
    out_type=jax.ShapeDtypeStruct((NC, HALF, D), jnp.float32),
    mesh=_mesh,
    scratch_types=[
        pltpu.VMEM_SHARED((HALF, D), jnp.float32),
        pltpu.VMEM_SHARED((HALF, D), jnp.float32),
        pltpu.VMEM((CAP,), jnp.int32),
        pltpu.VMEM((CAPC, K), jnp.int32),
        pltpu.VMEM((CAP,), jnp.float32),
        pltpu.VMEM((K, D), jnp.float32),
        pltpu.VMEM((K, D), jnp.float32),
        pltpu.SemaphoreType.DMA,
        pltpu.SemaphoreType.DMA,
    ],
    compiler_params=_sc_params,
)
def _mp_kernel(hs_hbm, srcl_hbm, dstl_hbm, ewl_hbm, out_hbm,
               table_sh, acc_sh, src_v, dst_v, ew_v, rows0, rows1,
               sem0, sem1):
    c = lax.axis_index("c")
    s = lax.axis_index("s")
    a0 = pl.multiple_of(s * ARW, 8)

    @pl.when(s < NS - 1)
    def _():
        pltpu.sync_copy(hs_hbm.at[pl.ds(pl.multiple_of(c * HALF + a0, 8), ARW)],
                        acc_sh.at[pl.ds(a0, ARW)])

    @pl.when(s == NS - 1)
    def _():
        pltpu.sync_copy(
            hs_hbm.at[pl.ds(pl.multiple_of(c * HALF + 15 * ARW, 8), ARW_LAST)],
            acc_sh.at[pl.ds(15 * ARW, ARW_LAST)])

    def scale(rows, base):
        def body(k, _):
            ewx = plsc.load_gather(ew_v, [jnp.full((16,), base + k, jnp.int32)])
            for cg in range(8):
                sl = pl.ds(cg * 16, 16)
                rows[k, sl] = rows[k, sl] * ewx
            return 0

        lax.fori_loop(0, K, body, 0, unroll=4)

    for p in range(2):
        srch = c if p == 0 else 1 - c
        toff = srch * HALF

        @pl.when(s < NS - 1)
        def _():
            pltpu.sync_copy(
                hs_hbm.at[pl.ds(pl.multiple_of(toff + a0, 8), ARW)],
                table_sh.at[pl.ds(a0, ARW)])

        @pl.when(s == NS - 1)
        def _():
            pltpu.sync_copy(
                hs_hbm.at[pl.ds(pl.multiple_of(toff + 15 * ARW, 8), ARW_LAST)],
                table_sh.at[pl.ds(15 * ARW, ARW_LAST)])

        plsc.subcore_barrier()

        b = 2 * srch + c
        for off in range(2):
            uw = 2 * s + off
            pltpu.sync_copy(srcl_hbm.at[uw, b], src_v)
            pltpu.sync_copy(ewl_hbm.at[uw, b], ew_v)
            pltpu.sync_copy(dstl_hbm.at[uw, b], dst_v)

            def issue(j, rows, sem):
                base = pl.multiple_of(j * K, 8)
                pltpu.async_copy(table_sh.at[src_v.at[pl.ds(base, K)]],
                                 rows, sem)

            def drain(j, rows, sem):
                base = pl.multiple_of(j * K, 8)
                pltpu.make_async_copy(table_sh.at[src_v.at[pl.ds(base, K)]],
                                      rows, sem).wait()

            issue(0, rows0, sem0)

            def chunk_body(q, _):
                j0 = 2 * q
                j1 = j0 + 1
                drain(j0, rows0, sem0)
                issue(j1, rows1, sem1)
                scale(rows0, j0 * K)
                pltpu.sync_copy(rows0, acc_sh.at[dst_v.at[j0]], add=True)
                drain(j1, rows1, sem1)

                @pl.when(j0 + 2 < CAPC)
                def _():
                    issue(j0 + 2, rows0, sem0)

                scale(rows1, j1 * K)
                pltpu.sync_copy(rows1, acc_sh.at[dst_v.at[j1]], add=True)
                return 0

            lax.fori_loop(0, CAPC // 2, chunk_body, 0)

        plsc.subcore_barrier()

    @pl.when(s < NS - 1)
    def _():
        pltpu.sync_copy(acc_sh.at[pl.ds(a0, ARW)],
                        out_hbm.at[c, pl.ds(a0, ARW)])

    @pl.when(s == NS - 1)
    def _():
        pltpu.sync_copy(acc_sh.at[pl.ds(15 * ARW, ARW_LAST)],
                        out_hbm.at[c, pl.ds(15 * ARW, ARW_LAST)])


def _d1_body(x_ref, w_ref, dinv_ref, out_ref):
    h = jnp.dot(x_ref[...], w_ref[...], preferred_element_type=jnp.float32)
    out_ref[...] = h * dinv_ref[...]


_d1 = pl.pallas_call(
    _d1_body,
    grid=(GRID,),
    in_specs=[
        pl.BlockSpec((ROWBLK, D), lambda i: (i, 0)),
        pl.BlockSpec((D, D), lambda i: (0, 0)),
        pl.BlockSpec((ROWBLK, 1), lambda i: (i, 0)),
    ],
    out_specs=pl.BlockSpec((ROWBLK, D), lambda i: (i, 0)),
    out_shape=jax.ShapeDtypeStruct((N_NODES, D), jnp.float32),
)


def _d2_body(p_ref, dinv_ref, b_ref, w_ref, out_ref):
    a = p_ref[...] * dinv_ref[...]
    y = jnp.maximum(a + b_ref[...], 0.0)
    h = jnp.dot(y, w_ref[...], preferred_element_type=jnp.float32)
    out_ref[...] = h * dinv_ref[...]


_d2 = pl.pallas_call(
    _d2_body,
    grid=(GRID,),
    in_specs=[
        pl.BlockSpec((ROWBLK, D), lambda i: (i, 0)),
        pl.BlockSpec((ROWBLK, 1), lambda i: (i, 0)),
        pl.BlockSpec((1, D), lambda i: (0, 0)),
        pl.BlockSpec((D, D), lambda i: (0, 0)),
    ],
    out_specs=pl.BlockSpec((ROWBLK, D), lambda i: (i, 0)),
    out_shape=jax.ShapeDtypeStruct((N_NODES, D), jnp.float32),
)


def _sigmoid(x):
    return 0.5 * (jnp.tanh(0.5 * x) + 1.0)


def _d3_body(p_ref, dinv_ref, b2_ref, wih_ref, bg_ref, wr_ref, br_ref,
             out_ref):
    a = p_ref[...] * dinv_ref[...]
    y = jnp.maximum(a + b2_ref[...], 0.0)
    g = jnp.dot(y, wih_ref[...], preferred_element_type=jnp.float32) + bg_ref[...]
    gi = _sigmoid(g[:, 0:D])
    gg = jnp.tanh(g[:, 2 * D:3 * D])
    go = _sigmoid(g[:, 3 * D:4 * D])
    h = go * jnp.tanh(gi * gg)
    out_ref[...] = (
        jnp.dot(h, wr_ref[...], preferred_element_type=jnp.float32) + br_ref[...]
    )


_d3 = pl.pallas_call(
    _d3_body,
    grid=(GRID,),
    in_specs=[
        pl.BlockSpec((ROWBLK, D), lambda i: (i, 0)),
        pl.BlockSpec((ROWBLK, 1), lambda i: (i, 0)),
        pl.BlockSpec((1, D), lambda i: (0, 0)),
        pl.BlockSpec((D, 4 * D), lambda i: (0, 0)),
        pl.BlockSpec((1, 4 * D), lambda i: (0, 0)),
        pl.BlockSpec((D, 1), lambda i: (0, 0)),
        pl.BlockSpec((1, 1), lambda i: (0, 0)),
    ],
    out_specs=pl.BlockSpec((ROWBLK, 1), lambda i: (i, 0)),
    out_shape=jax.ShapeDtypeStruct((N_NODES, 1), jnp.float32),
)


def kernel(x, edge_index, edge_weight, W1, b1, W2, b2, Wih, Whh, bih, bhh,
           Wr, br):
    src = edge_index[0].astype(jnp.int32)
    dst = edge_index[1].astype(jnp.int32)
    ew = edge_weight.astype(jnp.float32)
    pad = EPAD - N_EDGES
    src3 = jnp.pad(src, (0, pad)).reshape(NW, NCHUNK, K)
    dst3 = jnp.pad(dst, (0, pad)).reshape(NW, NCHUNK, K)
    ew3 = jnp.pad(ew, (0, pad)).reshape(NW, NCHUNK, K)

    degp = _deg_kernel(dst3, ew3)
    deg = jnp.sum(degp, axis=0) + 1.0
    dinv = lax.rsqrt(deg)[:, None]

    srcl, dstl, ewl = _part_kernel(src3, dst3, ew3)
    dstl4 = dstl.reshape(NW, 4, CAPC, K)

    hs1 = _d1(x, W1, dinv)
    a1 = _mp_kernel(hs1, srcl, dstl4, ewl).reshape(N_NODES, D)
    hs2 = _d2(a1, dinv, b1.reshape(1, D), W2)
    a2 = _mp_kernel(hs2, srcl, dstl4, ewl).reshape(N_NODES, D)
    out = _d3(a2, dinv, b2.reshape(1, D), Wih.T,
              (bih + bhh).reshape(1, 4 * D), Wr.T, br.reshape(1, 1))
    return out
